# baseline (speedup 1.0000x reference)
import functools
import math

import jax
import jax.numpy as jnp
from jax import lax
from jax.experimental import pallas as pl
from jax.experimental.pallas import tpu as pltpu
from jax.experimental.pallas import tpu_sc as plsc

N = 10000
E = 320000
IN_CH = 128
OUT_CH = 64

N_PAD = 10240
NW = 32
CHUNK = 128
EPW0 = 80
EPW1 = 80
EPW_CH = max(EPW0, EPW1)
E_PAD = 16 * (EPW0 + EPW1) * CHUNK
RPT = N_PAD // 16
SLAB = 128

_F32 = jnp.float32


D_SC = 64


def _make_sc_agg(n_tables, with_counts):
    mesh = plsc.VectorSubcoreMesh(core_axis_name="c", subcore_axis_name="s")
    outs = [jax.ShapeDtypeStruct((2, n_tables, N_PAD, D_SC), _F32)]
    if with_counts:
        outs.append(jax.ShapeDtypeStruct((2, N_PAD), _F32))

    scratch = [
        pltpu.VMEM((EPW_CH, CHUNK), jnp.int32),
        pltpu.VMEM((EPW_CH, CHUNK), jnp.int32),
        pltpu.VMEM((CHUNK, D_SC), _F32),
        pltpu.VMEM((SLAB, D_SC), _F32),
        pltpu.VMEM((CHUNK,), _F32),
        pltpu.VMEM((RPT,), _F32),
        pltpu.VMEM_SHARED((N_PAD, D_SC), _F32),
        pltpu.VMEM_SHARED((N_PAD,), _F32),
    ]

    def body(tbl, srcw, dstw, *rest):
        if with_counts:
            out, cnt_out = rest[0], rest[1]
            rest = rest[2:]
        else:
            out = rest[0]
            rest = rest[1:]
        src_v, dst_v, rows_v, slab_v, ones_v, crow_v, acc_s, cnt_s = rest

        cid = lax.axis_index("c")
        sid = lax.axis_index("s")
        wid = sid * 2 + cid

        pltpu.sync_copy(srcw.at[wid], src_v)
        pltpu.sync_copy(dstw.at[wid], dst_v)

        zero16 = jnp.zeros((16,), _F32)
        one16 = jnp.ones((16,), _F32)

        def zrow(r, carry):
            for c in range(D_SC // 16):
                slab_v[r, pl.ds(c * 16, 16)] = zero16
            return carry

        lax.fori_loop(0, SLAB, zrow, 0)
        for c in range(CHUNK // 16):
            ones_v[pl.ds(c * 16, 16)] = one16

        def zcnt(r, carry):
            crow_v[pl.ds(r * 16, 16)] = zero16
            return carry

        lax.fori_loop(0, RPT // 16, zcnt, 0)

        base0 = sid * RPT
        pltpu.sync_copy(crow_v, cnt_s.at[pl.ds(base0, RPT)])

        for h in range(n_tables):
            for k in range(RPT // SLAB):
                pltpu.sync_copy(slab_v,
                                acc_s.at[pl.ds(base0 + k * SLAB, SLAB)])
            plsc.subcore_barrier()

            count_now = with_counts and h == 0

            def step(ci, carry):
                pltpu.sync_copy(tbl.at[h].at[src_v.at[ci]], rows_v)
                pltpu.sync_copy(rows_v, acc_s.at[dst_v.at[ci]], add=True)
                if count_now:
                    pltpu.sync_copy(ones_v, cnt_s.at[dst_v.at[ci]], add=True)
                return carry

            lax.fori_loop(0, EPW_CH, step, 0)
            plsc.subcore_barrier()

            for k in range(RPT // SLAB):
                b = base0 + k * SLAB
                pltpu.sync_copy(acc_s.at[pl.ds(b, SLAB)], slab_v)
                pltpu.sync_copy(slab_v, out.at[cid, h, pl.ds(b, SLAB)])
            if h + 1 < n_tables:
                lax.fori_loop(0, SLAB, zrow, 0)

        if with_counts:
            pltpu.sync_copy(cnt_s.at[pl.ds(base0, RPT)], crow_v)
            pltpu.sync_copy(crow_v, cnt_out.at[cid, pl.ds(base0, RPT)])

    return pl.kernel(
        body, out_type=tuple(outs), mesh=mesh, scratch_types=scratch,
        compiler_params=pltpu.CompilerParams(use_tc_tiling_on_sc=False))


_R = 1024


def _mm0_body(x_ref, w_ref, p_ref, r_ref):
    y = jnp.dot(x_ref[...], w_ref[...], preferred_element_type=_F32)
    p_ref[0] = y[:, :D_SC]
    p_ref[1] = y[:, D_SC:IN_CH]
    r_ref[...] = y[:, IN_CH:]


def _mm0(xp, W0):
    return pl.pallas_call(
        _mm0_body,
        grid=(N_PAD // _R,),
        in_specs=[
            pl.BlockSpec((_R, IN_CH), lambda i: (i, 0)),
            pl.BlockSpec((IN_CH, 2 * IN_CH), lambda i: (0, 0)),
        ],
        out_specs=[
            pl.BlockSpec((2, _R, D_SC), lambda i: (0, i, 0)),
            pl.BlockSpec((_R, IN_CH), lambda i: (i, 0)),
        ],
        out_shape=[
            jax.ShapeDtypeStruct((2, N_PAD, D_SC), _F32),
            jax.ShapeDtypeStruct((N_PAD, IN_CH), _F32),
        ],
    )(xp, W0)


def _combine1_body(s0_ref, cnt_ref, r0_ref, x_ref, bl0_ref, g_ref, b_ref,
                   a_ref, w1_ref, bl1_ref, p1_ref, r1_ref):
    s = jnp.concatenate(
        [s0_ref[0, 0] + s0_ref[1, 0], s0_ref[0, 1] + s0_ref[1, 1]], axis=1)
    c = cnt_ref[0] + cnt_ref[1]
    mean = s / jnp.maximum(c, 1.0)[:, None]
    t = mean + bl0_ref[...] + r0_ref[...]
    scale = g_ref[...] * (1.0 / math.sqrt(1.0 + 1e-5))
    t = t * scale + b_ref[...]
    a = a_ref[0, 0]
    t = jnp.where(t >= 0, t, a * t)
    h = t + x_ref[...]
    y = jnp.dot(h, w1_ref[...], preferred_element_type=_F32)
    p1_ref[0] = y[:, :OUT_CH]
    r1_ref[...] = y[:, OUT_CH:] + bl1_ref[...]


def _combine1(sums0, cnt, r0, xp, bl0, gamma, beta, a11, W1, bl1):
    return pl.pallas_call(
        _combine1_body,
        grid=(N_PAD // _R,),
        in_specs=[
            pl.BlockSpec((2, 2, _R, D_SC), lambda i: (0, 0, i, 0)),
            pl.BlockSpec((2, _R), lambda i: (0, i)),
            pl.BlockSpec((_R, IN_CH), lambda i: (i, 0)),
            pl.BlockSpec((_R, IN_CH), lambda i: (i, 0)),
            pl.BlockSpec((IN_CH,), lambda i: (0,)),
            pl.BlockSpec((IN_CH,), lambda i: (0,)),
            pl.BlockSpec((IN_CH,), lambda i: (0,)),
            pl.BlockSpec(memory_space=pltpu.SMEM),
            pl.BlockSpec((IN_CH, 2 * OUT_CH), lambda i: (0, 0)),
            pl.BlockSpec((OUT_CH,), lambda i: (0,)),
        ],
        out_specs=[
            pl.BlockSpec((1, _R, OUT_CH), lambda i: (0, i, 0)),
            pl.BlockSpec((_R, OUT_CH), lambda i: (i, 0)),
        ],
        out_shape=[
            jax.ShapeDtypeStruct((1, N_PAD, OUT_CH), _F32),
            jax.ShapeDtypeStruct((N_PAD, OUT_CH), _F32),
        ],
    )(sums0, cnt, r0, xp, bl0, gamma, beta, a11, W1, bl1)


def _final_body(s1_ref, cnt_ref, r1_ref, out_ref):
    s = s1_ref[0, 0] + s1_ref[1, 0]
    c = cnt_ref[0] + cnt_ref[1]
    z = s / jnp.maximum(c, 1.0)[:, None] + r1_ref[...]
    m = jnp.max(z, axis=1, keepdims=True)
    e = jnp.exp(z - m)
    lse = jnp.log(jnp.sum(e, axis=1, keepdims=True)) + m
    out_ref[...] = z - lse


def _final(sums1, cnt, r1):
    return pl.pallas_call(
        _final_body,
        grid=(N_PAD // _R,),
        in_specs=[
            pl.BlockSpec((2, 1, _R, OUT_CH), lambda i: (0, 0, i, 0)),
            pl.BlockSpec((2, _R), lambda i: (0, i)),
            pl.BlockSpec((_R, OUT_CH), lambda i: (i, 0)),
        ],
        out_specs=pl.BlockSpec((_R, OUT_CH), lambda i: (i, 0)),
        out_shape=jax.ShapeDtypeStruct((N_PAD, OUT_CH), _F32),
    )(sums1, cnt, r1)


_sc_agg_cnt = _make_sc_agg(1, True)


@jax.jit
def _run(x, edge_index, Wl0, bl0, Wr0, Wl1, bl1, Wr1, gamma, beta, prelu_a):
    src = edge_index[0].astype(jnp.int32)
    dst = edge_index[1].astype(jnp.int32)
    pad = E_PAD - E
    fill = N + jnp.arange(pad, dtype=jnp.int32) % (N_PAD - N)

    def to_workers(flat):
        tot0 = 16 * EPW0 * CHUNK
        e0 = flat[:tot0].reshape(16, 1, EPW0, CHUNK)
        e0 = jnp.pad(e0, ((0, 0), (0, 0), (0, EPW_CH - EPW0), (0, 0)),
                     constant_values=N)
        e1 = flat[tot0:].reshape(16, 1, EPW1, CHUNK)
        e1 = jnp.pad(e1, ((0, 0), (0, 0), (0, EPW_CH - EPW1), (0, 0)),
                     constant_values=N)
        return jnp.concatenate([e0, e1], axis=1).reshape(NW, EPW_CH, CHUNK)

    srcw = to_workers(jnp.concatenate([src, fill]))
    dstw = to_workers(jnp.concatenate([dst, fill]))
    xp = jnp.pad(x, ((0, N_PAD - N), (0, 0)))
    W0 = jnp.concatenate([Wl0.T, Wr0.T], axis=1)
    W1 = jnp.concatenate([Wl1.T, Wr1.T], axis=1)
    a11 = prelu_a.reshape(1, 1)

    p0, r0 = _mm0(xp, W0)
    sums0a, cnt_a = _sc_agg_cnt(p0[0:1], srcw, dstw)
    sums0b, cnt_b = _sc_agg_cnt(p0[1:2], srcw, dstw)
    sums0 = jnp.concatenate([sums0a, sums0b], axis=1)
    cnt = 0.5 * (cnt_a + cnt_b)
    p1, r1 = _combine1(sums0, cnt, r0, xp, bl0, gamma, beta, a11, W1, bl1)
    sums1, cnt1 = _sc_agg_cnt(p1, srcw, dstw)
    out = _final(sums1, cnt1, r1)
    return out[:N]


def kernel(x, edge_index, Wl0, bl0, Wr0, Wl1, bl1, Wr1, gamma, beta, prelu_a):
    return _run(x, edge_index, Wl0, bl0, Wr0, Wl1, bl1, Wr1, gamma, beta,
                prelu_a)


# speedup vs baseline: 1.2487x; 1.2487x over previous
import functools
import math

import jax
import jax.numpy as jnp
from jax import lax
from jax.experimental import pallas as pl
from jax.experimental.pallas import tpu as pltpu
from jax.experimental.pallas import tpu_sc as plsc

N = 10000
E = 320000
IN_CH = 128
OUT_CH = 64

N_PAD = 10240
NW = 32
CHUNK = 128
EPW0 = 80
EPW1 = 80
EPW_CH = max(EPW0, EPW1)
E_PAD = 16 * (EPW0 + EPW1) * CHUNK
RPT = N_PAD // 16
SLAB = 128

_F32 = jnp.float32


D_SC = 64


def _make_sc_agg(n_tables, with_counts):
    mesh = plsc.VectorSubcoreMesh(core_axis_name="c", subcore_axis_name="s")
    outs = [jax.ShapeDtypeStruct((2, n_tables, N_PAD, D_SC), _F32)]
    if with_counts:
        outs.append(jax.ShapeDtypeStruct((2, N_PAD), _F32))

    scratch = [
        pltpu.VMEM((EPW_CH, CHUNK), jnp.int32),
        pltpu.VMEM((EPW_CH, CHUNK), jnp.int32),
        pltpu.VMEM((CHUNK, D_SC), _F32),
        pltpu.VMEM((CHUNK, D_SC), _F32),
        pltpu.VMEM((SLAB, D_SC), _F32),
        pltpu.VMEM((CHUNK,), _F32),
        pltpu.VMEM((RPT,), _F32),
        pltpu.VMEM_SHARED((N_PAD, D_SC), _F32),
        pltpu.VMEM_SHARED((N_PAD,), _F32),
        pltpu.SemaphoreType.DMA,
        pltpu.SemaphoreType.DMA,
    ]

    def body(tbl, srcw, dstw, *rest):
        if with_counts:
            out, cnt_out = rest[0], rest[1]
            rest = rest[2:]
        else:
            out = rest[0]
            rest = rest[1:]
        (src_v, dst_v, rows0_v, rows1_v, slab_v, ones_v, crow_v, acc_s,
         cnt_s, ssem, csem) = rest
        rows_bufs = (rows0_v, rows1_v)

        cid = lax.axis_index("c")
        sid = lax.axis_index("s")
        wid = sid * 2 + cid

        pltpu.sync_copy(srcw.at[wid], src_v)
        pltpu.sync_copy(dstw.at[wid], dst_v)

        zero16 = jnp.zeros((16,), _F32)
        one16 = jnp.ones((16,), _F32)

        def zrow(r, carry):
            for c in range(D_SC // 16):
                slab_v[r, pl.ds(c * 16, 16)] = zero16
            return carry

        lax.fori_loop(0, SLAB, zrow, 0)
        for c in range(CHUNK // 16):
            ones_v[pl.ds(c * 16, 16)] = one16

        def zcnt(r, carry):
            crow_v[pl.ds(r * 16, 16)] = zero16
            return carry

        lax.fori_loop(0, RPT // 16, zcnt, 0)

        base0 = sid * RPT
        pltpu.sync_copy(crow_v, cnt_s.at[pl.ds(base0, RPT)])

        for h in range(n_tables):
            for k in range(RPT // SLAB):
                pltpu.sync_copy(slab_v,
                                acc_s.at[pl.ds(base0 + k * SLAB, SLAB)])
            plsc.subcore_barrier()

            count_now = with_counts and h == 0

            def gather(ci, b):
                pltpu.sync_copy(tbl.at[h].at[src_v.at[ci]], rows_bufs[b])

            def s_start(ci, b):
                pltpu.async_copy(rows_bufs[b], acc_s.at[dst_v.at[ci]], ssem,
                                 add=True)

            def s_drain():
                pltpu.make_async_copy(rows_bufs[0], acc_s.at[dst_v.at[0]],
                                      ssem).wait()

            def c_start(ci):
                pltpu.async_copy(ones_v, cnt_s.at[dst_v.at[ci]], csem,
                                 add=True)

            def c_drain():
                pltpu.make_async_copy(ones_v, cnt_s.at[dst_v.at[0]],
                                      csem).wait()

            for b in range(2):
                gather(b, b)
                s_start(b, b)
                if count_now:
                    c_start(b)

            def pair(g, carry):
                for b in range(2):
                    ci = 2 * g + b
                    s_drain()
                    if count_now:
                        c_drain()
                    gather(ci, b)
                    s_start(ci, b)
                    if count_now:
                        c_start(ci)
                return carry

            lax.fori_loop(1, EPW_CH // 2, pair, 0)
            for _ in range(2):
                s_drain()
                if count_now:
                    c_drain()
            plsc.subcore_barrier()

            for k in range(RPT // SLAB):
                b = base0 + k * SLAB
                pltpu.sync_copy(acc_s.at[pl.ds(b, SLAB)], slab_v)
                pltpu.sync_copy(slab_v, out.at[cid, h, pl.ds(b, SLAB)])
            if h + 1 < n_tables:
                lax.fori_loop(0, SLAB, zrow, 0)

        if with_counts:
            pltpu.sync_copy(cnt_s.at[pl.ds(base0, RPT)], crow_v)
            pltpu.sync_copy(crow_v, cnt_out.at[cid, pl.ds(base0, RPT)])

    return pl.kernel(
        body, out_type=tuple(outs), mesh=mesh, scratch_types=scratch,
        compiler_params=pltpu.CompilerParams(use_tc_tiling_on_sc=False))


_R = 1024


def _mm0_body(x_ref, w_ref, p_ref, r_ref):
    y = jnp.dot(x_ref[...], w_ref[...], preferred_element_type=_F32)
    p_ref[0] = y[:, :D_SC]
    p_ref[1] = y[:, D_SC:IN_CH]
    r_ref[...] = y[:, IN_CH:]


def _mm0(xp, W0):
    return pl.pallas_call(
        _mm0_body,
        grid=(N_PAD // _R,),
        in_specs=[
            pl.BlockSpec((_R, IN_CH), lambda i: (i, 0)),
            pl.BlockSpec((IN_CH, 2 * IN_CH), lambda i: (0, 0)),
        ],
        out_specs=[
            pl.BlockSpec((2, _R, D_SC), lambda i: (0, i, 0)),
            pl.BlockSpec((_R, IN_CH), lambda i: (i, 0)),
        ],
        out_shape=[
            jax.ShapeDtypeStruct((2, N_PAD, D_SC), _F32),
            jax.ShapeDtypeStruct((N_PAD, IN_CH), _F32),
        ],
    )(xp, W0)


def _combine1_body(s0_ref, cnt_ref, r0_ref, x_ref, bl0_ref, g_ref, b_ref,
                   a_ref, w1_ref, bl1_ref, p1_ref, r1_ref):
    s = jnp.concatenate(
        [s0_ref[0, 0] + s0_ref[1, 0], s0_ref[0, 1] + s0_ref[1, 1]], axis=1)
    c = cnt_ref[0] + cnt_ref[1]
    mean = s / jnp.maximum(c, 1.0)[:, None]
    t = mean + bl0_ref[...] + r0_ref[...]
    scale = g_ref[...] * (1.0 / math.sqrt(1.0 + 1e-5))
    t = t * scale + b_ref[...]
    a = a_ref[0, 0]
    t = jnp.where(t >= 0, t, a * t)
    h = t + x_ref[...]
    y = jnp.dot(h, w1_ref[...], preferred_element_type=_F32)
    p1_ref[0] = y[:, :OUT_CH]
    r1_ref[...] = y[:, OUT_CH:] + bl1_ref[...]


def _combine1(sums0, cnt, r0, xp, bl0, gamma, beta, a11, W1, bl1):
    return pl.pallas_call(
        _combine1_body,
        grid=(N_PAD // _R,),
        in_specs=[
            pl.BlockSpec((2, 2, _R, D_SC), lambda i: (0, 0, i, 0)),
            pl.BlockSpec((2, _R), lambda i: (0, i)),
            pl.BlockSpec((_R, IN_CH), lambda i: (i, 0)),
            pl.BlockSpec((_R, IN_CH), lambda i: (i, 0)),
            pl.BlockSpec((IN_CH,), lambda i: (0,)),
            pl.BlockSpec((IN_CH,), lambda i: (0,)),
            pl.BlockSpec((IN_CH,), lambda i: (0,)),
            pl.BlockSpec(memory_space=pltpu.SMEM),
            pl.BlockSpec((IN_CH, 2 * OUT_CH), lambda i: (0, 0)),
            pl.BlockSpec((OUT_CH,), lambda i: (0,)),
        ],
        out_specs=[
            pl.BlockSpec((1, _R, OUT_CH), lambda i: (0, i, 0)),
            pl.BlockSpec((_R, OUT_CH), lambda i: (i, 0)),
        ],
        out_shape=[
            jax.ShapeDtypeStruct((1, N_PAD, OUT_CH), _F32),
            jax.ShapeDtypeStruct((N_PAD, OUT_CH), _F32),
        ],
    )(sums0, cnt, r0, xp, bl0, gamma, beta, a11, W1, bl1)


def _final_body(s1_ref, cnt_ref, r1_ref, out_ref):
    s = s1_ref[0, 0] + s1_ref[1, 0]
    c = cnt_ref[0] + cnt_ref[1]
    z = s / jnp.maximum(c, 1.0)[:, None] + r1_ref[...]
    m = jnp.max(z, axis=1, keepdims=True)
    e = jnp.exp(z - m)
    lse = jnp.log(jnp.sum(e, axis=1, keepdims=True)) + m
    out_ref[...] = z - lse


def _final(sums1, cnt, r1):
    return pl.pallas_call(
        _final_body,
        grid=(N_PAD // _R,),
        in_specs=[
            pl.BlockSpec((2, 1, _R, OUT_CH), lambda i: (0, 0, i, 0)),
            pl.BlockSpec((2, _R), lambda i: (0, i)),
            pl.BlockSpec((_R, OUT_CH), lambda i: (i, 0)),
        ],
        out_specs=pl.BlockSpec((_R, OUT_CH), lambda i: (i, 0)),
        out_shape=jax.ShapeDtypeStruct((N_PAD, OUT_CH), _F32),
    )(sums1, cnt, r1)


_sc_agg_cnt = _make_sc_agg(1, True)


@jax.jit
def _run(x, edge_index, Wl0, bl0, Wr0, Wl1, bl1, Wr1, gamma, beta, prelu_a):
    src = edge_index[0].astype(jnp.int32)
    dst = edge_index[1].astype(jnp.int32)
    pad = E_PAD - E
    fill = N + jnp.arange(pad, dtype=jnp.int32) % (N_PAD - N)

    def to_workers(flat):
        tot0 = 16 * EPW0 * CHUNK
        e0 = flat[:tot0].reshape(16, 1, EPW0, CHUNK)
        e0 = jnp.pad(e0, ((0, 0), (0, 0), (0, EPW_CH - EPW0), (0, 0)),
                     constant_values=N)
        e1 = flat[tot0:].reshape(16, 1, EPW1, CHUNK)
        e1 = jnp.pad(e1, ((0, 0), (0, 0), (0, EPW_CH - EPW1), (0, 0)),
                     constant_values=N)
        return jnp.concatenate([e0, e1], axis=1).reshape(NW, EPW_CH, CHUNK)

    srcw = to_workers(jnp.concatenate([src, fill]))
    dstw = to_workers(jnp.concatenate([dst, fill]))
    xp = jnp.pad(x, ((0, N_PAD - N), (0, 0)))
    W0 = jnp.concatenate([Wl0.T, Wr0.T], axis=1)
    W1 = jnp.concatenate([Wl1.T, Wr1.T], axis=1)
    a11 = prelu_a.reshape(1, 1)

    p0, r0 = _mm0(xp, W0)
    sums0a, cnt_a = _sc_agg_cnt(p0[0:1], srcw, dstw)
    sums0b, cnt_b = _sc_agg_cnt(p0[1:2], srcw, dstw)
    sums0 = jnp.concatenate([sums0a, sums0b], axis=1)
    cnt = 0.5 * (cnt_a + cnt_b)
    p1, r1 = _combine1(sums0, cnt, r0, xp, bl0, gamma, beta, a11, W1, bl1)
    sums1, cnt1 = _sc_agg_cnt(p1, srcw, dstw)
    out = _final(sums1, cnt1, r1)
    return out[:N]


def kernel(x, edge_index, Wl0, bl0, Wr0, Wl1, bl1, Wr1, gamma, beta, prelu_a):
    return _run(x, edge_index, Wl0, bl0, Wr0, Wl1, bl1, Wr1, gamma, beta,
                prelu_a)


# speedup vs baseline: 1.5140x; 1.2125x over previous
import functools
import math

import jax
import jax.numpy as jnp
from jax import lax
from jax.experimental import pallas as pl
from jax.experimental.pallas import tpu as pltpu
from jax.experimental.pallas import tpu_sc as plsc

N = 10000
E = 320000
IN_CH = 128
OUT_CH = 64

N_PAD = 10240
NW = 32
CHUNK = 128
EPW0 = 80
EPW1 = 80
EPW_CH = max(EPW0, EPW1)
E_PAD = 16 * (EPW0 + EPW1) * CHUNK
RPT = N_PAD // 16
SLAB = 128

_F32 = jnp.float32


D_SC = 64


def _make_sc_agg(n_tables, with_counts):
    mesh = plsc.VectorSubcoreMesh(core_axis_name="c", subcore_axis_name="s")
    outs = [jax.ShapeDtypeStruct((2, n_tables, N_PAD, D_SC), _F32)]
    if with_counts:
        outs.append(jax.ShapeDtypeStruct((2, N_PAD), _F32))

    scratch = [
        pltpu.VMEM((EPW_CH, CHUNK), jnp.int32),
        pltpu.VMEM((EPW_CH, CHUNK), jnp.int32),
        pltpu.VMEM((CHUNK, D_SC), _F32),
        pltpu.VMEM((CHUNK, D_SC), _F32),
        pltpu.VMEM((CHUNK, D_SC), _F32),
        pltpu.VMEM((CHUNK, D_SC), _F32),
        pltpu.VMEM((SLAB, D_SC), _F32),
        pltpu.VMEM((CHUNK,), _F32),
        pltpu.VMEM((RPT,), _F32),
        pltpu.VMEM_SHARED((N_PAD, D_SC), _F32),
        pltpu.VMEM_SHARED((N_PAD,), _F32),
        pltpu.SemaphoreType.DMA,
        pltpu.SemaphoreType.DMA,
        pltpu.SemaphoreType.DMA,
        pltpu.SemaphoreType.DMA,
        pltpu.SemaphoreType.DMA,
        pltpu.SemaphoreType.DMA,
    ]

    def body(tbl, srcw, dstw, *rest):
        if with_counts:
            out, cnt_out = rest[0], rest[1]
            rest = rest[2:]
        else:
            out = rest[0]
            rest = rest[1:]
        (src_v, dst_v, rows0_v, rows1_v, rows2_v, rows3_v, slab_v, ones_v,
         crow_v, acc_s, cnt_s, ssem, csem, gsem0, gsem1, gsem2,
         gsem3) = rest
        rows_bufs = (rows0_v, rows1_v, rows2_v, rows3_v)
        gsems = (gsem0, gsem1, gsem2, gsem3)

        cid = lax.axis_index("c")
        sid = lax.axis_index("s")
        wid = sid * 2 + cid

        pltpu.sync_copy(srcw.at[wid], src_v)
        pltpu.sync_copy(dstw.at[wid], dst_v)

        zero16 = jnp.zeros((16,), _F32)
        one16 = jnp.ones((16,), _F32)

        def zrow(r, carry):
            for c in range(D_SC // 16):
                slab_v[r, pl.ds(c * 16, 16)] = zero16
            return carry

        lax.fori_loop(0, SLAB, zrow, 0)
        for c in range(CHUNK // 16):
            ones_v[pl.ds(c * 16, 16)] = one16

        def zcnt(r, carry):
            crow_v[pl.ds(r * 16, 16)] = zero16
            return carry

        lax.fori_loop(0, RPT // 16, zcnt, 0)

        base0 = sid * RPT
        pltpu.sync_copy(crow_v, cnt_s.at[pl.ds(base0, RPT)])

        for h in range(n_tables):
            for k in range(RPT // SLAB):
                pltpu.sync_copy(slab_v,
                                acc_s.at[pl.ds(base0 + k * SLAB, SLAB)])
            plsc.subcore_barrier()

            count_now = with_counts and h == 0

            def g_start(ci, b):
                pltpu.async_copy(tbl.at[h].at[src_v.at[ci]], rows_bufs[b],
                                 gsems[b])

            def g_wait(b):
                pltpu.make_async_copy(tbl.at[h].at[src_v.at[0]],
                                      rows_bufs[b], gsems[b]).wait()

            def s_start(ci, b):
                pltpu.async_copy(rows_bufs[b], acc_s.at[dst_v.at[ci]], ssem,
                                 add=True)

            def s_drain():
                pltpu.make_async_copy(rows_bufs[0], acc_s.at[dst_v.at[0]],
                                      ssem).wait()

            def c_start(ci):
                pltpu.async_copy(ones_v, cnt_s.at[dst_v.at[ci]], csem,
                                 add=True)

            def c_drain():
                pltpu.make_async_copy(ones_v, cnt_s.at[dst_v.at[0]],
                                      csem).wait()

            g_start(0, 0)
            g_start(1, 1)
            NG4 = EPW_CH // 4
            for b in range(4):
                g_wait(b)
                s_start(b, b)
                if count_now:
                    c_start(b)
                if b >= 2:
                    s_drain()
                    if count_now:
                        c_drain()
                g_start(b + 2, (b + 2) % 4)

            def quad(g, carry):
                for b in range(4):
                    ci = 4 * g + b
                    g_wait(b)
                    s_start(ci, b)
                    if count_now:
                        c_start(ci)
                    s_drain()
                    if count_now:
                        c_drain()
                    g_start(ci + 2, (b + 2) % 4)
                return carry

            lax.fori_loop(1, NG4 - 1, quad, 0)

            for b in range(4):
                ci = (NG4 - 1) * 4 + b
                g_wait(b)
                s_start(ci, b)
                if count_now:
                    c_start(ci)
                s_drain()
                if count_now:
                    c_drain()
                if b < 2:
                    g_start(ci + 2, (b + 2) % 4)

            for _ in range(2):
                s_drain()
                if count_now:
                    c_drain()
            plsc.subcore_barrier()

            for k in range(RPT // SLAB):
                b = base0 + k * SLAB
                pltpu.sync_copy(acc_s.at[pl.ds(b, SLAB)], slab_v)
                pltpu.sync_copy(slab_v, out.at[cid, h, pl.ds(b, SLAB)])
            if h + 1 < n_tables:
                lax.fori_loop(0, SLAB, zrow, 0)

        if with_counts:
            pltpu.sync_copy(cnt_s.at[pl.ds(base0, RPT)], crow_v)
            pltpu.sync_copy(crow_v, cnt_out.at[cid, pl.ds(base0, RPT)])

    return pl.kernel(
        body, out_type=tuple(outs), mesh=mesh, scratch_types=scratch,
        compiler_params=pltpu.CompilerParams(use_tc_tiling_on_sc=False))


_R = 1024


def _mm0_body(x_ref, w_ref, p_ref, r_ref):
    y = jnp.dot(x_ref[...], w_ref[...], preferred_element_type=_F32)
    p_ref[0] = y[:, :D_SC]
    p_ref[1] = y[:, D_SC:IN_CH]
    r_ref[...] = y[:, IN_CH:]


def _mm0(xp, W0):
    return pl.pallas_call(
        _mm0_body,
        grid=(N_PAD // _R,),
        in_specs=[
            pl.BlockSpec((_R, IN_CH), lambda i: (i, 0)),
            pl.BlockSpec((IN_CH, 2 * IN_CH), lambda i: (0, 0)),
        ],
        out_specs=[
            pl.BlockSpec((2, _R, D_SC), lambda i: (0, i, 0)),
            pl.BlockSpec((_R, IN_CH), lambda i: (i, 0)),
        ],
        out_shape=[
            jax.ShapeDtypeStruct((2, N_PAD, D_SC), _F32),
            jax.ShapeDtypeStruct((N_PAD, IN_CH), _F32),
        ],
    )(xp, W0)


def _combine1_body(s0_ref, cnt_ref, r0_ref, x_ref, bl0_ref, g_ref, b_ref,
                   a_ref, w1_ref, bl1_ref, p1_ref, r1_ref):
    s = jnp.concatenate(
        [s0_ref[0, 0] + s0_ref[1, 0], s0_ref[0, 1] + s0_ref[1, 1]], axis=1)
    c = cnt_ref[0] + cnt_ref[1]
    mean = s / jnp.maximum(c, 1.0)[:, None]
    t = mean + bl0_ref[...] + r0_ref[...]
    scale = g_ref[...] * (1.0 / math.sqrt(1.0 + 1e-5))
    t = t * scale + b_ref[...]
    a = a_ref[0, 0]
    t = jnp.where(t >= 0, t, a * t)
    h = t + x_ref[...]
    y = jnp.dot(h, w1_ref[...], preferred_element_type=_F32)
    p1_ref[0] = y[:, :OUT_CH]
    r1_ref[...] = y[:, OUT_CH:] + bl1_ref[...]


def _combine1(sums0, cnt, r0, xp, bl0, gamma, beta, a11, W1, bl1):
    return pl.pallas_call(
        _combine1_body,
        grid=(N_PAD // _R,),
        in_specs=[
            pl.BlockSpec((2, 2, _R, D_SC), lambda i: (0, 0, i, 0)),
            pl.BlockSpec((2, _R), lambda i: (0, i)),
            pl.BlockSpec((_R, IN_CH), lambda i: (i, 0)),
            pl.BlockSpec((_R, IN_CH), lambda i: (i, 0)),
            pl.BlockSpec((IN_CH,), lambda i: (0,)),
            pl.BlockSpec((IN_CH,), lambda i: (0,)),
            pl.BlockSpec((IN_CH,), lambda i: (0,)),
            pl.BlockSpec(memory_space=pltpu.SMEM),
            pl.BlockSpec((IN_CH, 2 * OUT_CH), lambda i: (0, 0)),
            pl.BlockSpec((OUT_CH,), lambda i: (0,)),
        ],
        out_specs=[
            pl.BlockSpec((1, _R, OUT_CH), lambda i: (0, i, 0)),
            pl.BlockSpec((_R, OUT_CH), lambda i: (i, 0)),
        ],
        out_shape=[
            jax.ShapeDtypeStruct((1, N_PAD, OUT_CH), _F32),
            jax.ShapeDtypeStruct((N_PAD, OUT_CH), _F32),
        ],
    )(sums0, cnt, r0, xp, bl0, gamma, beta, a11, W1, bl1)


def _final_body(s1_ref, cnt_ref, r1_ref, out_ref):
    s = s1_ref[0, 0] + s1_ref[1, 0]
    c = cnt_ref[0] + cnt_ref[1]
    z = s / jnp.maximum(c, 1.0)[:, None] + r1_ref[...]
    m = jnp.max(z, axis=1, keepdims=True)
    e = jnp.exp(z - m)
    lse = jnp.log(jnp.sum(e, axis=1, keepdims=True)) + m
    out_ref[...] = z - lse


def _final(sums1, cnt, r1):
    return pl.pallas_call(
        _final_body,
        grid=(N_PAD // _R,),
        in_specs=[
            pl.BlockSpec((2, 1, _R, OUT_CH), lambda i: (0, 0, i, 0)),
            pl.BlockSpec((2, _R), lambda i: (0, i)),
            pl.BlockSpec((_R, OUT_CH), lambda i: (i, 0)),
        ],
        out_specs=pl.BlockSpec((_R, OUT_CH), lambda i: (i, 0)),
        out_shape=jax.ShapeDtypeStruct((N_PAD, OUT_CH), _F32),
    )(sums1, cnt, r1)


_sc_agg_cnt = _make_sc_agg(1, True)


@jax.jit
def _run(x, edge_index, Wl0, bl0, Wr0, Wl1, bl1, Wr1, gamma, beta, prelu_a):
    src = edge_index[0].astype(jnp.int32)
    dst = edge_index[1].astype(jnp.int32)
    pad = E_PAD - E
    fill = N + jnp.arange(pad, dtype=jnp.int32) % (N_PAD - N)

    def to_workers(flat):
        tot0 = 16 * EPW0 * CHUNK
        e0 = flat[:tot0].reshape(16, 1, EPW0, CHUNK)
        e0 = jnp.pad(e0, ((0, 0), (0, 0), (0, EPW_CH - EPW0), (0, 0)),
                     constant_values=N)
        e1 = flat[tot0:].reshape(16, 1, EPW1, CHUNK)
        e1 = jnp.pad(e1, ((0, 0), (0, 0), (0, EPW_CH - EPW1), (0, 0)),
                     constant_values=N)
        return jnp.concatenate([e0, e1], axis=1).reshape(NW, EPW_CH, CHUNK)

    srcw = to_workers(jnp.concatenate([src, fill]))
    dstw = to_workers(jnp.concatenate([dst, fill]))
    xp = jnp.pad(x, ((0, N_PAD - N), (0, 0)))
    W0 = jnp.concatenate([Wl0.T, Wr0.T], axis=1)
    W1 = jnp.concatenate([Wl1.T, Wr1.T], axis=1)
    a11 = prelu_a.reshape(1, 1)

    p0, r0 = _mm0(xp, W0)
    sums0a, cnt_a = _sc_agg_cnt(p0[0:1], srcw, dstw)
    sums0b, cnt_b = _sc_agg_cnt(p0[1:2], srcw, dstw)
    sums0 = jnp.concatenate([sums0a, sums0b], axis=1)
    cnt = 0.5 * (cnt_a + cnt_b)
    p1, r1 = _combine1(sums0, cnt, r0, xp, bl0, gamma, beta, a11, W1, bl1)
    sums1, cnt1 = _sc_agg_cnt(p1, srcw, dstw)
    out = _final(sums1, cnt1, r1)
    return out[:N]


def kernel(x, edge_index, Wl0, bl0, Wr0, Wl1, bl1, Wr1, gamma, beta, prelu_a):
    return _run(x, edge_index, Wl0, bl0, Wr0, Wl1, bl1, Wr1, gamma, beta,
                prelu_a)
